# Initial kernel scaffold; baseline (speedup 1.0000x reference)
import jax
import jax.numpy as jnp
from jax.experimental import pallas as pl


def kernel(obj_fmaps, obj_logits, rel_inds, vr, boxes_per_cls, params):
    raise NotImplementedError("write your pallas kernel here")


# speedup vs baseline: 32.0769x; 32.0769x over previous
import functools

import jax
import jax.numpy as jnp
from jax import lax
from jax.experimental import pallas as pl
from jax.experimental.pallas import tpu as pltpu
from jax.experimental.pallas import tpu_sc as plsc

N = 1000
C = 150
CP = 160
NP = 1024
R = 256
HID = 256
OBJ_DIM = 4096
NUM_REL = 51
TSTEPS = 3
NMS_T = 0.3

NW = 32
ROWS_PER_W = CP // NW
GB = 512 // NW


CHUNK = 128


def _nms_body(xcat_ref, ps_ref, out_ref, keep_ref):
    keep_ref[...] = jnp.ones((CP, NP), jnp.float32)
    sub128 = lax.broadcasted_iota(jnp.int32, (CHUNK, 1), 0)

    for ci in range(NP // CHUNK):
        base = ci * CHUNK
        span = NP - base
        hi = min(N - base, CHUNK)
        if hi <= 0:
            break
        lane = lax.broadcasted_iota(jnp.int32, (CP, span), 1)
        xs = xcat_ref[:, base:]
        x1 = xs[0 * CP:1 * CP]
        y1 = xs[1 * CP:2 * CP]
        x2 = xs[2 * CP:3 * CP]
        y2 = xs[3 * CP:4 * CP]
        area = (x2 - x1) * (y2 - y1)
        xc = xcat_ref[:, base:base + CHUNK]

        def body(j, carry, xc=xc, x1=x1, y1=y1, x2=x2, y2=y2, area=area,
                 lane=lane, base=base, span=span):
            oh = (sub128 == j).astype(jnp.float32)
            cs = jax.lax.dot(xc, oh, precision=jax.lax.Precision.HIGHEST,
                             preferred_element_type=jnp.float32)
            kc = keep_ref[:, base:base + CHUNK]
            ki = jax.lax.dot(kc, oh, precision=jax.lax.Precision.HIGHEST,
                             preferred_element_type=jnp.float32)
            x1i = cs[0 * CP:1 * CP]
            y1i = cs[1 * CP:2 * CP]
            x2i = cs[2 * CP:3 * CP]
            y2i = cs[3 * CP:4 * CP]
            ai = (x2i - x1i) * (y2i - y1i)
            xx1 = jnp.maximum(x1i, x1)
            yy1 = jnp.maximum(y1i, y1)
            xx2 = jnp.minimum(x2i, x2)
            yy2 = jnp.minimum(y2i, y2)
            inter = jnp.maximum(xx2 - xx1, 0.0) * jnp.maximum(yy2 - yy1, 0.0)
            iou = inter / (ai + area - inter + 1e-8)
            sup = ((iou > NMS_T) & (lane > j)).astype(jnp.float32) * ki
            keep_ref[:, base:] = keep_ref[:, base:] * (1.0 - sup)
            return carry

        lax.fori_loop(0, hi, body, 0)

    out_ref[...] = keep_ref[...] * ps_ref[...]


def _nms_call(xcat, ps):
    return pl.pallas_call(
        _nms_body,
        out_shape=jax.ShapeDtypeStruct((CP, NP), jnp.float32),
        scratch_shapes=[pltpu.VMEM((CP, NP), jnp.float32)],
    )(xcat, ps)


def _argmax_body(val_ref, out_ref):
    val = val_ref[...]
    m = jnp.max(val, axis=0, keepdims=True)
    cidx = lax.broadcasted_iota(jnp.int32, (CP, NP), 0)
    cand = jnp.where(val == m, cidx, CP)
    out_ref[...] = jnp.min(cand, axis=0, keepdims=True) + 1


def _argmax_call(val):
    return pl.pallas_call(
        _argmax_body,
        out_shape=jax.ShapeDtypeStruct((1, NP), jnp.int32),
    )(val)


SC_ROWS = CP * NP // 128 // NW


@functools.lru_cache(maxsize=None)
def _scatter_call():
    mesh = plsc.VectorSubcoreMesh(core_axis_name="c", subcore_axis_name="s")

    @functools.partial(
        pl.kernel,
        out_type=jax.ShapeDtypeStruct((CP * NP,), jnp.float32),
        mesh=mesh,
        scratch_types=[
            pltpu.VMEM((SC_ROWS, 128), jnp.float32),
            pltpu.VMEM((SC_ROWS, 128), jnp.int32),
        ],
    )
    def k(val_hbm, idx_hbm, out_hbm, val_v, idx_v):
        wid = lax.axis_index("c") * 16 + lax.axis_index("s")
        base = wid * SC_ROWS
        pltpu.sync_copy(val_hbm.at[pl.ds(base, SC_ROWS)], val_v)
        pltpu.sync_copy(idx_hbm.at[pl.ds(base, SC_ROWS)], idx_v)

        def body(j, carry):
            pltpu.sync_copy(val_v.at[j], out_hbm.at[idx_v.at[j]])
            return carry

        lax.fori_loop(0, SC_ROWS, body, 0)

    return k


@functools.lru_cache(maxsize=None)
def _gather_call():
    mesh = plsc.VectorSubcoreMesh(core_axis_name="c", subcore_axis_name="s")

    @functools.partial(
        pl.kernel,
        out_type=jax.ShapeDtypeStruct((2 * R, OBJ_DIM), jnp.float32),
        mesh=mesh,
        scratch_types=[
            pltpu.VMEM((GB,), jnp.int32),
            pltpu.VMEM((GB, OBJ_DIM), jnp.float32),
            pltpu.SemaphoreType.DMA,
        ],
    )
    def k(table_hbm, idx_hbm, out_hbm, idx_v, rows_v, sem):
        wid = lax.axis_index("c") * 16 + lax.axis_index("s")
        base = wid * GB
        pltpu.sync_copy(idx_hbm.at[pl.ds(base, GB)], idx_v)
        pltpu.async_copy(table_hbm.at[idx_v], rows_v, sem).wait()
        pltpu.sync_copy(rows_v, out_hbm.at[pl.ds(base, GB)])

    return k


def _ggnn_body(g_ref, vr_ref, wo_ref, bo_ref, wr_ref, br_ref,
               w3_ref, u3_ref, b3_ref, w4_ref, u4_ref, b4_ref,
               w5_ref, u5_ref, b5_ref, woa_ref, wob_ref, bout_ref,
               wsc_ref, bsc_ref, out_ref):
    def dot(a, b):
        return jnp.dot(a, b, preferred_element_type=jnp.float32)

    p = dot(g_ref[...], wo_ref[...]) + bo_ref[...]
    vrp = dot(vr_ref[...], wr_ref[...]) + br_ref[...]
    h = jnp.concatenate([p, vrp], axis=0)
    c = jnp.float32(1.0 / NUM_REL)
    for _ in range(TSTEPS):
        h0 = h[:R]
        h1 = h[R:2 * R]
        e = h[2 * R:]
        m = h0 * c + h1 * c
        wside = jnp.concatenate([e, e, m], axis=0)
        z = jax.nn.sigmoid(dot(wside, w3_ref[...]) + dot(h, u3_ref[...])
                           + b3_ref[...])
        r = jax.nn.sigmoid(dot(wside, w4_ref[...]) + dot(h, u4_ref[...])
                           + b4_ref[...])
        hh = jnp.tanh(dot(wside, w5_ref[...]) + dot(r * h, u5_ref[...])
                      + b5_ref[...])
        h = (1.0 - z) * h + z * hh
    e = h[2 * R:]
    out = jax.nn.relu(dot(e, woa_ref[...]) + dot(vrp, wob_ref[...])
                      + bout_ref[...])
    rel = jnp.sum(out * wsc_ref[...], axis=1, keepdims=True) + bsc_ref[...]
    out_ref[...] = jnp.broadcast_to(rel, (R, NUM_REL))


def _ggnn_call(*args):
    return pl.pallas_call(
        _ggnn_body,
        out_shape=jax.ShapeDtypeStruct((R, NUM_REL), jnp.float32),
    )(*args)


def kernel(obj_fmaps, obj_logits, rel_inds, vr, boxes_per_cls, params):
    probs = jax.nn.softmax(obj_logits, axis=1)
    scores = probs[:, 1:].T
    order = jnp.argsort(-scores, axis=1).astype(jnp.int32)
    boxes = jnp.transpose(boxes_per_cls[:, 1:, :], (1, 0, 2))
    sbox = jnp.take_along_axis(boxes, order[:, :, None], axis=1)
    ps = jnp.take_along_axis(scores, order, axis=1)

    pad = ((0, CP - C), (0, NP - N))
    x1 = jnp.pad(sbox[..., 0], pad)
    y1 = jnp.pad(sbox[..., 1], pad)
    x2 = jnp.pad(sbox[..., 2], pad)
    y2 = jnp.pad(sbox[..., 3], pad)
    psp = jnp.pad(ps, pad)
    lane = jnp.arange(NP, dtype=jnp.int32)
    ordp = jnp.concatenate(
        [jnp.concatenate([order, jnp.broadcast_to(lane[N:], (C, NP - N))], axis=1),
         jnp.broadcast_to(lane, (CP - C, NP))], axis=0)

    xcat = jnp.concatenate([x1, y1, x2, y2], axis=0)
    val_sorted = _nms_call(xcat, psp)
    gidx = jnp.arange(CP, dtype=jnp.int32)[:, None] * NP + ordp
    val_orig = _scatter_call()(
        val_sorted.reshape(-1, 128), gidx.reshape(-1, 128)
    ).reshape(CP, NP)
    obj_preds = _argmax_call(val_orig)[0, :N]

    idx = jnp.concatenate([rel_inds[:, 1], rel_inds[:, 2]]).astype(jnp.int32)
    g = _gather_call()(obj_fmaps, idx)

    wo, bo = params['obj_proj']
    wr, br = params['rel_proj']

    def fold(name):
        w, bw = params[name + '_w']
        u, bu = params[name + '_u']
        return w[:HID] + w[HID:], u, (bw + bu)[None, :]

    w3, u3, b3 = fold('eq3')
    w4, u4, b4 = fold('eq4')
    w5, u5, b5 = fold('eq5')
    wout, bout = params['fc_output']
    wsc, bsc = params['fc_score']

    rel_dists = _ggnn_call(g, vr, wo, bo[None, :], wr, br[None, :],
                           w3, u3, b3, w4, u4, b4, w5, u5, b5,
                           wout[:HID], wout[HID:], bout[None, :],
                           wsc.T, bsc.reshape(1, 1))
    return (obj_logits, obj_preds, rel_dists)
